# baseline (speedup 1.0000x reference)
import dataclasses
import functools

import jax
import jax.numpy as jnp
from jax import lax
from jax.experimental import pallas as pl
from jax.experimental.pallas import tpu as pltpu
from jax.experimental.pallas import tpu_sc as plsc

N = 10000
D = 128
DH = D // 2
N_PAD = 10240
NW = 32
CH = 128
NBUF = 4
KCH = 1
NDR = N_PAD // D
ROWS_PER_TILE = N_PAD // 16

_mesh = plsc.VectorSubcoreMesh(core_axis_name="c", subcore_axis_name="s")

_sc_params = pltpu.CompilerParams()
if "needs_layout_passes" in pltpu.CompilerParams.__dataclass_fields__:
  _sc_params = dataclasses.replace(_sc_params, needs_layout_passes=False)
if "use_tc_tiling_on_sc" in pltpu.CompilerParams.__dataclass_fields__:
  _sc_params = dataclasses.replace(_sc_params, use_tc_tiling_on_sc=False)


def _make_sc_deg(n_chunks):
  scratch = [
      pltpu.VMEM((n_chunks, CH), jnp.int32),
      pltpu.VMEM((NDR, D), jnp.float32),
      pltpu.VMEM((1, NDR), jnp.int32),
      pltpu.VMEM_SHARED((NDR, D), jnp.float32),
  ]

  def body(dstr_hbm, deg_hbm, dst_v, deg_v, ridx_v, deg_sh):
    cid = lax.axis_index("c")
    sid = lax.axis_index("s")
    wid = cid * 16 + sid

    zero16 = jnp.zeros((16,), jnp.float32)

    @pl.loop(0, NDR)
    def _(r):
      @pl.loop(0, D, step=16)
      def _(c):
        deg_v[r, pl.ds(c, 16)] = zero16

    @pl.when(sid == 0)
    def _():
      pltpu.sync_copy(deg_v, deg_sh)
    iota16 = lax.iota(jnp.int32, 16)
    for j in range(NDR // 16):
      ridx_v[0, pl.ds(j * 16, 16)] = iota16 + j * 16

    plsc.subcore_barrier()

    pltpu.sync_copy(dstr_hbm.at[wid], dst_v)
    ones16 = jnp.ones((16,), jnp.float32)

    @pl.loop(0, n_chunks)
    def _(i):
      for j in range(CH // 16):
        idx = dst_v[i, pl.ds(j * 16, 16)]
        plsc.addupdate_scatter(
            deg_v,
            [lax.shift_right_logical(idx, 7), lax.bitwise_and(idx, 127)],
            ones16)

    pltpu.sync_copy(deg_v, deg_sh.at[ridx_v.at[0]], add=True)
    plsc.subcore_barrier()

    @pl.when(sid == 0)
    def _():
      pltpu.sync_copy(deg_sh, deg_hbm.at[pl.ds(cid * NDR, NDR)])

  return pl.kernel(
      body,
      out_type=jax.ShapeDtypeStruct((2 * NDR, D), jnp.float32),
      mesh=_mesh, scratch_types=scratch, compiler_params=_sc_params)


def _make_sc_agg(n_chunks):
  ng = 2 * n_chunks // KCH
  assert ng % NBUF == 0
  gw = KCH * CH
  scratch = [
      pltpu.VMEM((ng, gw), jnp.int32),
      pltpu.VMEM((ng, gw), jnp.int32),
  ]
  scratch += [pltpu.VMEM((gw, DH), jnp.float32) for _ in range(NBUF)]
  scratch += [
      pltpu.VMEM_SHARED((N_PAD, DH), jnp.float32),
  ]
  scratch += [pltpu.SemaphoreType.DMA for _ in range(2 * NBUF)]

  def body(h_hbm, srcr_hbm, dstr_hbm, acc_hbm, src_v, dst_v, *rest):
    bufs = rest[:NBUF]
    acc_sh = rest[NBUF]
    gsems = rest[NBUF + 1:NBUF + 1 + NBUF]
    ssems = rest[NBUF + 1 + NBUF:]
    cid = lax.axis_index("c")
    sid = lax.axis_index("s")

    zero16 = jnp.zeros((16,), jnp.float32)
    base = sid * ROWS_PER_TILE

    @pl.loop(0, gw)
    def _(r):
      @pl.loop(0, DH, step=16)
      def _(c):
        bufs[0][r, pl.ds(c, 16)] = zero16

    @pl.loop(0, ROWS_PER_TILE // gw)
    def _(k):
      pltpu.sync_copy(bufs[0], acc_sh.at[pl.ds(base + k * gw, gw)])

    plsc.subcore_barrier()

    nhalf = ng // 2

    @pl.loop(0, 2)
    def _(p):
      pltpu.sync_copy(srcr_hbm.at[cid, 2 * sid + p],
                      src_v.at[pl.ds(p * nhalf, nhalf)])
      pltpu.sync_copy(dstr_hbm.at[2 * sid + p],
                      dst_v.at[pl.ds(p * nhalf, nhalf)])

    def g_start(i, k):
      pltpu.async_copy(h_hbm.at[src_v.at[i]], bufs[k], gsems[k])

    def g_wait(i, k):
      pltpu.make_async_copy(h_hbm.at[src_v.at[i]], bufs[k], gsems[k]).wait()

    def s_start(i, k):
      pltpu.async_copy(bufs[k], acc_sh.at[dst_v.at[i]], ssems[k], add=True)

    def s_wait(i, k):
      pltpu.make_async_copy(bufs[k], acc_sh.at[dst_v.at[i]], ssems[k]).wait()

    @pl.loop(0, ng, step=NBUF)
    def _(i):
      for k in range(NBUF):
        g_start(i + k, k)
      for k in range(NBUF):
        g_wait(i + k, k)
        s_start(i + k, k)
      for k in range(NBUF):
        s_wait(i + k, k)

    plsc.subcore_barrier()

    pltpu.sync_copy(acc_sh.at[pl.ds(base, ROWS_PER_TILE)],
                    acc_hbm.at[pl.ds(cid * N_PAD + base, ROWS_PER_TILE)])

  return pl.kernel(
      body,
      out_type=jax.ShapeDtypeStruct((2 * N_PAD, DH), jnp.float32),
      mesh=_mesh, scratch_types=scratch, compiler_params=_sc_params)


_DOT = functools.partial(
    lax.dot_general,
    dimension_numbers=(((1,), (0,)), ((), ())),
    preferred_element_type=jnp.float32,
    precision=lax.Precision.HIGHEST)


def _k_in_body(x_ref, w_ref, b_ref, o_ref):
  t = jnp.tanh(_DOT(x_ref[...], w_ref[...]) + b_ref[...])
  o_ref[0] = t[:, :DH]
  o_ref[1] = t[:, DH:]


def _layer_math(h_ref, acc_ref, d0_ref, d1_ref, ws_ref, b_ref, wn_ref):
  deg = jnp.maximum(d0_ref[...] + d1_ref[...], 1.0)
  ws = ws_ref[...]
  wn = wn_ref[...]
  t = _DOT(h_ref[0], ws[:DH]) + _DOT(h_ref[1], ws[DH:])
  t += _DOT(acc_ref[0] / deg, wn[:DH]) + _DOT(acc_ref[1] / deg, wn[DH:])
  return jnp.maximum(t + b_ref[...], 0.0)


def _k_layer_body(h_ref, acc_ref, d0_ref, d1_ref, ws_ref, b_ref, wn_ref,
                  o_ref):
  r = _layer_math(h_ref, acc_ref, d0_ref, d1_ref, ws_ref, b_ref, wn_ref)
  o_ref[0] = r[:, :DH]
  o_ref[1] = r[:, DH:]


def _k_last_body(h_ref, acc_ref, d0_ref, d1_ref, ws_ref, b_ref, wn_ref,
                 o_ref):
  o_ref[...] = _layer_math(h_ref, acc_ref, d0_ref, d1_ref, ws_ref, b_ref,
                           wn_ref)


_BLK = 1024
_GRID = N_PAD // _BLK
_row_spec = pl.BlockSpec((_BLK, D), lambda i: (i, 0))
_split_spec = pl.BlockSpec((2, _BLK, DH), lambda i: (0, i, 0))
_w_spec = pl.BlockSpec((D, D), lambda i: (0, 0))
_b_spec = pl.BlockSpec((1, D), lambda i: (0, 0))
_split_sds = jax.ShapeDtypeStruct((2, N_PAD, DH), jnp.float32)

_k_in = pl.pallas_call(
    _k_in_body,
    grid=(_GRID,),
    in_specs=[_row_spec, _w_spec, _b_spec],
    out_specs=_split_spec,
    out_shape=_split_sds)

_layer_in_specs = [
    _split_spec,
    _split_spec,
    pl.BlockSpec((_BLK, 1), lambda i: (i, 0)),
    pl.BlockSpec((_BLK, 1), lambda i: (i + _GRID, 0)),
    _w_spec, _b_spec, _w_spec,
]

_k_layer = pl.pallas_call(
    _k_layer_body,
    grid=(_GRID,),
    in_specs=_layer_in_specs,
    out_specs=_split_spec,
    out_shape=_split_sds)

_k_last = pl.pallas_call(
    _k_last_body,
    grid=(_GRID,),
    in_specs=_layer_in_specs,
    out_specs=_row_spec,
    out_shape=jax.ShapeDtypeStruct((N_PAD, D), jnp.float32))


def kernel(x, edge_index, W_in, b_in, W_self1, b_self1, W_neigh1,
           W_self2, b_self2, W_neigh2):
  E = edge_index.shape[1]
  n_chunks = -(-E // (NW * CH))
  if n_chunks % 2:
    n_chunks += 1
  e_pad = NW * CH * n_chunks - E

  xp = jnp.zeros((N_PAD, D), jnp.float32).at[:N].set(x)
  src = edge_index[0]
  dst = edge_index[1]
  if e_pad:
    ar = jnp.arange(e_pad, dtype=jnp.int32)
    src = jnp.concatenate([src, ar % N])
    dst = jnp.concatenate([dst, N + ar % (N_PAD - N)])
  srcr = src.reshape(NW, n_chunks // KCH, KCH * CH)
  dstr = dst.reshape(NW, n_chunks // KCH, KCH * CH)
  srcr2 = jnp.stack([srcr, srcr + N_PAD])
  srcr2, dstr = lax.optimization_barrier((srcr2, dstr))

  sc_deg = _make_sc_deg(n_chunks)
  sc_agg = _make_sc_agg(n_chunks)

  b_in2 = b_in.reshape(1, D)
  b1 = b_self1.reshape(1, D)
  b2 = b_self2.reshape(1, D)

  deg = sc_deg(dstr.reshape(NW, n_chunks, CH))
  degf = deg.reshape(2 * N_PAD, 1)
  h0 = _k_in(xp, W_in, b_in2)
  acc1 = sc_agg(h0.reshape(2 * N_PAD, DH), srcr2, dstr)
  h1 = _k_layer(h0, acc1.reshape(2, N_PAD, DH), degf, degf, W_self1, b1,
                W_neigh1)
  acc2 = sc_agg(h1.reshape(2 * N_PAD, DH), srcr2, dstr)
  h2 = _k_last(h1, acc2.reshape(2, N_PAD, DH), degf, degf, W_self2, b2,
               W_neigh2)
  return h2[:N]


# speedup vs baseline: 1.1727x; 1.1727x over previous
import dataclasses
import functools

import jax
import jax.numpy as jnp
from jax import lax
from jax.experimental import pallas as pl
from jax.experimental.pallas import tpu as pltpu
from jax.experimental.pallas import tpu_sc as plsc

N = 10000
D = 128
DH = D // 2
N_PAD = 10240
NW = 32
CH = 128
NBUF = 4
KCH = 1
NDR = N_PAD // D
ROWS_PER_TILE = N_PAD // 16

_mesh = plsc.VectorSubcoreMesh(core_axis_name="c", subcore_axis_name="s")

_sc_params = pltpu.CompilerParams()
if "needs_layout_passes" in pltpu.CompilerParams.__dataclass_fields__:
  _sc_params = dataclasses.replace(_sc_params, needs_layout_passes=False)
if "use_tc_tiling_on_sc" in pltpu.CompilerParams.__dataclass_fields__:
  _sc_params = dataclasses.replace(_sc_params, use_tc_tiling_on_sc=False)


def _make_sc_deg(n_chunks):
  scratch = [
      pltpu.VMEM((n_chunks, CH), jnp.int32),
      pltpu.VMEM((NDR, D), jnp.float32),
      pltpu.VMEM((1, NDR), jnp.int32),
      pltpu.VMEM_SHARED((NDR, D), jnp.float32),
  ]

  def body(dstr_hbm, deg_hbm, dst_v, deg_v, ridx_v, deg_sh):
    cid = lax.axis_index("c")
    sid = lax.axis_index("s")
    wid = cid * 16 + sid

    zero16 = jnp.zeros((16,), jnp.float32)

    @pl.loop(0, NDR)
    def _(r):
      @pl.loop(0, D, step=16)
      def _(c):
        deg_v[r, pl.ds(c, 16)] = zero16

    @pl.when(sid == 0)
    def _():
      pltpu.sync_copy(deg_v, deg_sh)
    iota16 = lax.iota(jnp.int32, 16)
    for j in range(NDR // 16):
      ridx_v[0, pl.ds(j * 16, 16)] = iota16 + j * 16

    plsc.subcore_barrier()

    pltpu.sync_copy(dstr_hbm.at[wid], dst_v)
    ones16 = jnp.ones((16,), jnp.float32)

    @pl.loop(0, n_chunks)
    def _(i):
      for j in range(CH // 16):
        idx = dst_v[i, pl.ds(j * 16, 16)]
        plsc.addupdate_scatter(
            deg_v,
            [lax.shift_right_logical(idx, 7), lax.bitwise_and(idx, 127)],
            ones16)

    pltpu.sync_copy(deg_v, deg_sh.at[ridx_v.at[0]], add=True)
    plsc.subcore_barrier()

    @pl.when(sid == 0)
    def _():
      pltpu.sync_copy(deg_sh, deg_hbm.at[pl.ds(cid * NDR, NDR)])

  return pl.kernel(
      body,
      out_type=jax.ShapeDtypeStruct((2 * NDR, D), jnp.float32),
      mesh=_mesh, scratch_types=scratch, compiler_params=_sc_params)


def _make_sc_agg(n_chunks):
  ng = 2 * n_chunks // KCH
  assert ng % NBUF == 0
  gw = KCH * CH
  scratch = [
      pltpu.VMEM((ng, gw), jnp.int32),
      pltpu.VMEM((ng, gw), jnp.int32),
  ]
  scratch += [pltpu.VMEM((gw, DH), jnp.float32) for _ in range(NBUF)]
  scratch += [
      pltpu.VMEM_SHARED((N_PAD, DH), jnp.float32),
  ]
  scratch += [pltpu.SemaphoreType.DMA for _ in range(2 * NBUF)]

  def body(h_hbm, srcr_hbm, dstr_hbm, acc_hbm, src_v, dst_v, *rest):
    bufs = rest[:NBUF]
    acc_sh = rest[NBUF]
    gsems = rest[NBUF + 1:NBUF + 1 + NBUF]
    ssems = rest[NBUF + 1 + NBUF:]
    cid = lax.axis_index("c")
    sid = lax.axis_index("s")

    zero16 = jnp.zeros((16,), jnp.float32)
    base = sid * ROWS_PER_TILE

    @pl.loop(0, gw)
    def _(r):
      @pl.loop(0, DH, step=16)
      def _(c):
        bufs[0][r, pl.ds(c, 16)] = zero16

    @pl.loop(0, ROWS_PER_TILE // gw)
    def _(k):
      pltpu.sync_copy(bufs[0], acc_sh.at[pl.ds(base + k * gw, gw)])

    plsc.subcore_barrier()

    nhalf = ng // 2

    @pl.loop(0, 2)
    def _(p):
      pltpu.sync_copy(srcr_hbm.at[cid, 2 * sid + p],
                      src_v.at[pl.ds(p * nhalf, nhalf)])
      pltpu.sync_copy(dstr_hbm.at[2 * sid + p],
                      dst_v.at[pl.ds(p * nhalf, nhalf)])

    def g_start(i, k):
      pltpu.async_copy(h_hbm.at[src_v.at[i]], bufs[k], gsems[k])

    def g_wait(i, k):
      pltpu.make_async_copy(h_hbm.at[src_v.at[i]], bufs[k], gsems[k]).wait()

    def s_start(i, k):
      pltpu.async_copy(bufs[k], acc_sh.at[dst_v.at[i]], ssems[k], add=True)

    def s_wait(i, k):
      pltpu.make_async_copy(bufs[k], acc_sh.at[dst_v.at[i]], ssems[k]).wait()

    @pl.loop(0, ng, step=NBUF)
    def _(i):
      for k in range(NBUF):
        @pl.when(i >= NBUF)
        def _():
          s_wait(i - NBUF + k, k)
        g_start(i + k, k)
      for k in range(NBUF):
        g_wait(i + k, k)
        s_start(i + k, k)

    for k in range(NBUF):
      s_wait(ng - NBUF + k, k)

    plsc.subcore_barrier()

    pltpu.sync_copy(acc_sh.at[pl.ds(base, ROWS_PER_TILE)],
                    acc_hbm.at[pl.ds(cid * N_PAD + base, ROWS_PER_TILE)])

  return pl.kernel(
      body,
      out_type=jax.ShapeDtypeStruct((2 * N_PAD, DH), jnp.float32),
      mesh=_mesh, scratch_types=scratch, compiler_params=_sc_params)


_DOT = functools.partial(
    lax.dot_general,
    dimension_numbers=(((1,), (0,)), ((), ())),
    preferred_element_type=jnp.float32,
    precision=lax.Precision.HIGHEST)


def _k_in_body(x_ref, w_ref, b_ref, o_ref):
  t = jnp.tanh(_DOT(x_ref[...], w_ref[...]) + b_ref[...])
  o_ref[0] = t[:, :DH]
  o_ref[1] = t[:, DH:]


def _layer_math(h_ref, acc_ref, d0_ref, d1_ref, ws_ref, b_ref, wn_ref):
  deg = jnp.maximum(d0_ref[...] + d1_ref[...], 1.0)
  ws = ws_ref[...]
  wn = wn_ref[...]
  t = _DOT(h_ref[0], ws[:DH]) + _DOT(h_ref[1], ws[DH:])
  t += _DOT(acc_ref[0] / deg, wn[:DH]) + _DOT(acc_ref[1] / deg, wn[DH:])
  return jnp.maximum(t + b_ref[...], 0.0)


def _k_layer_body(h_ref, acc_ref, d0_ref, d1_ref, ws_ref, b_ref, wn_ref,
                  o_ref):
  r = _layer_math(h_ref, acc_ref, d0_ref, d1_ref, ws_ref, b_ref, wn_ref)
  o_ref[0] = r[:, :DH]
  o_ref[1] = r[:, DH:]


def _k_last_body(h_ref, acc_ref, d0_ref, d1_ref, ws_ref, b_ref, wn_ref,
                 o_ref):
  o_ref[...] = _layer_math(h_ref, acc_ref, d0_ref, d1_ref, ws_ref, b_ref,
                           wn_ref)


_BLK = 1024
_GRID = N_PAD // _BLK
_row_spec = pl.BlockSpec((_BLK, D), lambda i: (i, 0))
_split_spec = pl.BlockSpec((2, _BLK, DH), lambda i: (0, i, 0))
_w_spec = pl.BlockSpec((D, D), lambda i: (0, 0))
_b_spec = pl.BlockSpec((1, D), lambda i: (0, 0))
_split_sds = jax.ShapeDtypeStruct((2, N_PAD, DH), jnp.float32)

_k_in = pl.pallas_call(
    _k_in_body,
    grid=(_GRID,),
    in_specs=[_row_spec, _w_spec, _b_spec],
    out_specs=_split_spec,
    out_shape=_split_sds)

_layer_in_specs = [
    _split_spec,
    _split_spec,
    pl.BlockSpec((_BLK, 1), lambda i: (i, 0)),
    pl.BlockSpec((_BLK, 1), lambda i: (i + _GRID, 0)),
    _w_spec, _b_spec, _w_spec,
]

_k_layer = pl.pallas_call(
    _k_layer_body,
    grid=(_GRID,),
    in_specs=_layer_in_specs,
    out_specs=_split_spec,
    out_shape=_split_sds)

_k_last = pl.pallas_call(
    _k_last_body,
    grid=(_GRID,),
    in_specs=_layer_in_specs,
    out_specs=_row_spec,
    out_shape=jax.ShapeDtypeStruct((N_PAD, D), jnp.float32))


def kernel(x, edge_index, W_in, b_in, W_self1, b_self1, W_neigh1,
           W_self2, b_self2, W_neigh2):
  E = edge_index.shape[1]
  n_chunks = -(-E // (NW * CH))
  if n_chunks % 2:
    n_chunks += 1
  e_pad = NW * CH * n_chunks - E

  xp = jnp.zeros((N_PAD, D), jnp.float32).at[:N].set(x)
  src = edge_index[0]
  dst = edge_index[1]
  if e_pad:
    ar = jnp.arange(e_pad, dtype=jnp.int32)
    src = jnp.concatenate([src, ar % N])
    dst = jnp.concatenate([dst, N + ar % (N_PAD - N)])
  srcr = src.reshape(NW, n_chunks // KCH, KCH * CH)
  dstr = dst.reshape(NW, n_chunks // KCH, KCH * CH)
  srcr2 = jnp.stack([srcr, srcr + N_PAD])
  srcr2, dstr = lax.optimization_barrier((srcr2, dstr))

  sc_deg = _make_sc_deg(n_chunks)
  sc_agg = _make_sc_agg(n_chunks)

  b_in2 = b_in.reshape(1, D)
  b1 = b_self1.reshape(1, D)
  b2 = b_self2.reshape(1, D)

  deg = sc_deg(dstr.reshape(NW, n_chunks, CH))
  degf = deg.reshape(2 * N_PAD, 1)
  h0 = _k_in(xp, W_in, b_in2)
  acc1 = sc_agg(h0.reshape(2 * N_PAD, DH), srcr2, dstr)
  h1 = _k_layer(h0, acc1.reshape(2, N_PAD, DH), degf, degf, W_self1, b1,
                W_neigh1)
  acc2 = sc_agg(h1.reshape(2 * N_PAD, DH), srcr2, dstr)
  h2 = _k_last(h1, acc2.reshape(2, N_PAD, DH), degf, degf, W_self2, b2,
               W_neigh2)
  return h2[:N]


# speedup vs baseline: 1.2344x; 1.0526x over previous
import dataclasses
import functools

import jax
import jax.numpy as jnp
from jax import lax
from jax.experimental import pallas as pl
from jax.experimental.pallas import tpu as pltpu
from jax.experimental.pallas import tpu_sc as plsc

N = 10000
D = 128
DH = D // 2
N_PAD = 10240
NW = 32
CH = 128
NBUF = 4
KCH = 1
NDR = N_PAD // D
ROWS_PER_TILE = N_PAD // 16

_mesh = plsc.VectorSubcoreMesh(core_axis_name="c", subcore_axis_name="s")

_sc_params = pltpu.CompilerParams()
if "needs_layout_passes" in pltpu.CompilerParams.__dataclass_fields__:
  _sc_params = dataclasses.replace(_sc_params, needs_layout_passes=False)
if "use_tc_tiling_on_sc" in pltpu.CompilerParams.__dataclass_fields__:
  _sc_params = dataclasses.replace(_sc_params, use_tc_tiling_on_sc=False)


def _make_sc_deg(n_chunks):
  scratch = [
      pltpu.VMEM((n_chunks, CH), jnp.int32),
      pltpu.VMEM((NDR, D), jnp.float32),
      pltpu.VMEM((1, NDR), jnp.int32),
      pltpu.VMEM_SHARED((NDR, D), jnp.float32),
  ]

  def body(dstr_hbm, deg_hbm, dst_v, deg_v, ridx_v, deg_sh):
    cid = lax.axis_index("c")
    sid = lax.axis_index("s")
    wid = cid * 16 + sid

    zero16 = jnp.zeros((16,), jnp.float32)

    @pl.loop(0, NDR)
    def _(r):
      @pl.loop(0, D, step=16)
      def _(c):
        deg_v[r, pl.ds(c, 16)] = zero16

    @pl.when(sid == 0)
    def _():
      pltpu.sync_copy(deg_v, deg_sh)
    iota16 = lax.iota(jnp.int32, 16)
    for j in range(NDR // 16):
      ridx_v[0, pl.ds(j * 16, 16)] = iota16 + j * 16

    plsc.subcore_barrier()

    pltpu.sync_copy(dstr_hbm.at[wid], dst_v)
    ones16 = jnp.ones((16,), jnp.float32)

    @pl.loop(0, n_chunks)
    def _(i):
      for j in range(CH // 16):
        idx = dst_v[i, pl.ds(j * 16, 16)]
        plsc.addupdate_scatter(
            deg_v,
            [lax.shift_right_logical(idx, 7), lax.bitwise_and(idx, 127)],
            ones16)

    pltpu.sync_copy(deg_v, deg_sh.at[ridx_v.at[0]], add=True)
    plsc.subcore_barrier()

    @pl.when(sid == 0)
    def _():
      pltpu.sync_copy(deg_sh, deg_hbm.at[pl.ds(cid * NDR, NDR)])

  return pl.kernel(
      body,
      out_type=jax.ShapeDtypeStruct((2 * NDR, D), jnp.float32),
      mesh=_mesh, scratch_types=scratch, compiler_params=_sc_params)


def _make_sc_agg(n_chunks):
  ng = 2 * n_chunks // KCH
  assert ng % NBUF == 0
  gw = KCH * CH
  scratch = [
      pltpu.VMEM((ng, gw), jnp.int32),
      pltpu.VMEM((ng, gw), jnp.int32),
  ]
  scratch += [pltpu.VMEM((gw, DH), jnp.float32) for _ in range(NBUF)]
  scratch += [
      pltpu.VMEM_SHARED((N_PAD, DH), jnp.float32),
  ]
  scratch += [pltpu.SemaphoreType.DMA for _ in range(2 * NBUF)]

  def body(h_hbm, srcr_hbm, dstr_hbm, acc_hbm, src_v, dst_v, *rest):
    bufs = rest[:NBUF]
    acc_sh = rest[NBUF]
    gsems = rest[NBUF + 1:NBUF + 1 + NBUF]
    ssems = rest[NBUF + 1 + NBUF:]
    cid = lax.axis_index("c")
    sid = lax.axis_index("s")

    zero16 = jnp.zeros((16,), jnp.float32)
    base = sid * ROWS_PER_TILE

    @pl.loop(0, gw)
    def _(r):
      @pl.loop(0, DH, step=16)
      def _(c):
        bufs[0][r, pl.ds(c, 16)] = zero16

    @pl.loop(0, ROWS_PER_TILE // gw)
    def _(k):
      pltpu.sync_copy(bufs[0], acc_sh.at[pl.ds(base + k * gw, gw)])

    plsc.subcore_barrier()

    nhalf = ng // 2

    @pl.loop(0, 2)
    def _(p):
      pltpu.sync_copy(srcr_hbm.at[cid, 2 * sid + p],
                      src_v.at[pl.ds(p * nhalf, nhalf)])
      pltpu.sync_copy(dstr_hbm.at[2 * sid + p],
                      dst_v.at[pl.ds(p * nhalf, nhalf)])

    def g_start(i, k):
      pltpu.async_copy(h_hbm.at[src_v.at[i]], bufs[k], gsems[k])

    def g_wait(i, k):
      pltpu.make_async_copy(h_hbm.at[src_v.at[i]], bufs[k], gsems[k]).wait()

    def s_start(i, k):
      pltpu.async_copy(bufs[k], acc_sh.at[dst_v.at[i]], ssems[k], add=True)

    def s_wait(i, k):
      pltpu.make_async_copy(bufs[k], acc_sh.at[dst_v.at[i]], ssems[k]).wait()

    @pl.loop(0, ng, step=NBUF)
    def _(i):
      for k in range(NBUF):
        @pl.when(i >= NBUF)
        def _():
          s_wait(i - NBUF + k, k)
        g_start(i + k, k)
      for k in range(NBUF):
        g_wait(i + k, k)
        s_start(i + k, k)

    for k in range(NBUF):
      s_wait(ng - NBUF + k, k)

    plsc.subcore_barrier()

    pltpu.sync_copy(acc_sh.at[pl.ds(base, ROWS_PER_TILE)],
                    acc_hbm.at[pl.ds(cid * N_PAD + base, ROWS_PER_TILE)])

  return pl.kernel(
      body,
      out_type=jax.ShapeDtypeStruct((2 * N_PAD, DH), jnp.float32),
      mesh=_mesh, scratch_types=scratch, compiler_params=_sc_params)


_DOT = functools.partial(
    lax.dot_general,
    dimension_numbers=(((1,), (0,)), ((), ())),
    preferred_element_type=jnp.float32)


def _k_in_body(x_ref, w_ref, b_ref, o_ref):
  t = jnp.tanh(_DOT(x_ref[...], w_ref[...]) + b_ref[...])
  o_ref[0] = t[:, :DH]
  o_ref[1] = t[:, DH:]


def _layer_math(h_ref, acc_ref, d0_ref, d1_ref, ws_ref, b_ref, wn_ref):
  deg = jnp.maximum(d0_ref[...] + d1_ref[...], 1.0)
  ws = ws_ref[...]
  wn = wn_ref[...]
  t = _DOT(h_ref[0], ws[:DH]) + _DOT(h_ref[1], ws[DH:])
  t += _DOT(acc_ref[0] / deg, wn[:DH]) + _DOT(acc_ref[1] / deg, wn[DH:])
  return jnp.maximum(t + b_ref[...], 0.0)


def _k_layer_body(h_ref, acc_ref, d0_ref, d1_ref, ws_ref, b_ref, wn_ref,
                  o_ref):
  r = _layer_math(h_ref, acc_ref, d0_ref, d1_ref, ws_ref, b_ref, wn_ref)
  o_ref[0] = r[:, :DH]
  o_ref[1] = r[:, DH:]


def _k_last_body(h_ref, acc_ref, d0_ref, d1_ref, ws_ref, b_ref, wn_ref,
                 o_ref):
  o_ref[...] = _layer_math(h_ref, acc_ref, d0_ref, d1_ref, ws_ref, b_ref,
                           wn_ref)


_BLK = 1024
_GRID = N_PAD // _BLK
_row_spec = pl.BlockSpec((_BLK, D), lambda i: (i, 0))
_split_spec = pl.BlockSpec((2, _BLK, DH), lambda i: (0, i, 0))
_w_spec = pl.BlockSpec((D, D), lambda i: (0, 0))
_b_spec = pl.BlockSpec((1, D), lambda i: (0, 0))
_split_sds = jax.ShapeDtypeStruct((2, N_PAD, DH), jnp.float32)

_k_in = pl.pallas_call(
    _k_in_body,
    grid=(_GRID,),
    in_specs=[_row_spec, _w_spec, _b_spec],
    out_specs=_split_spec,
    out_shape=_split_sds)

_layer_in_specs = [
    _split_spec,
    _split_spec,
    pl.BlockSpec((_BLK, 1), lambda i: (i, 0)),
    pl.BlockSpec((_BLK, 1), lambda i: (i + _GRID, 0)),
    _w_spec, _b_spec, _w_spec,
]

_k_layer = pl.pallas_call(
    _k_layer_body,
    grid=(_GRID,),
    in_specs=_layer_in_specs,
    out_specs=_split_spec,
    out_shape=_split_sds)

_k_last = pl.pallas_call(
    _k_last_body,
    grid=(_GRID,),
    in_specs=_layer_in_specs,
    out_specs=_row_spec,
    out_shape=jax.ShapeDtypeStruct((N_PAD, D), jnp.float32))


def kernel(x, edge_index, W_in, b_in, W_self1, b_self1, W_neigh1,
           W_self2, b_self2, W_neigh2):
  E = edge_index.shape[1]
  n_chunks = -(-E // (NW * CH))
  if n_chunks % 2:
    n_chunks += 1
  e_pad = NW * CH * n_chunks - E

  xp = jnp.zeros((N_PAD, D), jnp.float32).at[:N].set(x)
  src = edge_index[0]
  dst = edge_index[1]
  if e_pad:
    ar = jnp.arange(e_pad, dtype=jnp.int32)
    src = jnp.concatenate([src, ar % N])
    dst = jnp.concatenate([dst, N + ar % (N_PAD - N)])
  srcr = src.reshape(NW, n_chunks // KCH, KCH * CH)
  dstr = dst.reshape(NW, n_chunks // KCH, KCH * CH)
  srcr2 = jnp.stack([srcr, srcr + N_PAD])
  srcr2, dstr = lax.optimization_barrier((srcr2, dstr))

  sc_deg = _make_sc_deg(n_chunks)
  sc_agg = _make_sc_agg(n_chunks)

  b_in2 = b_in.reshape(1, D)
  b1 = b_self1.reshape(1, D)
  b2 = b_self2.reshape(1, D)

  deg = sc_deg(dstr.reshape(NW, n_chunks, CH))
  degf = deg.reshape(2 * N_PAD, 1)
  h0 = _k_in(xp, W_in, b_in2)
  acc1 = sc_agg(h0.reshape(2 * N_PAD, DH), srcr2, dstr)
  h1 = _k_layer(h0, acc1.reshape(2, N_PAD, DH), degf, degf, W_self1, b1,
                W_neigh1)
  acc2 = sc_agg(h1.reshape(2 * N_PAD, DH), srcr2, dstr)
  h2 = _k_last(h1, acc2.reshape(2, N_PAD, DH), degf, degf, W_self2, b2,
               W_neigh2)
  return h2[:N]


# speedup vs baseline: 1.2406x; 1.0050x over previous
import dataclasses
import functools

import jax
import jax.numpy as jnp
from jax import lax
from jax.experimental import pallas as pl
from jax.experimental.pallas import tpu as pltpu
from jax.experimental.pallas import tpu_sc as plsc

N = 10000
D = 128
DH = D // 2
N_PAD = 10240
NW = 32
CH = 128
NBUF = 4
KCH = 1
NDR = N_PAD // D
ROWS_PER_TILE = N_PAD // 16

_mesh = plsc.VectorSubcoreMesh(core_axis_name="c", subcore_axis_name="s")

_sc_params = pltpu.CompilerParams()
if "needs_layout_passes" in pltpu.CompilerParams.__dataclass_fields__:
  _sc_params = dataclasses.replace(_sc_params, needs_layout_passes=False)
if "use_tc_tiling_on_sc" in pltpu.CompilerParams.__dataclass_fields__:
  _sc_params = dataclasses.replace(_sc_params, use_tc_tiling_on_sc=False)


def _make_sc_deg(n_chunks):
  scratch = [
      pltpu.VMEM((n_chunks, CH), jnp.int32),
      pltpu.VMEM((NDR, D), jnp.float32),
      pltpu.VMEM((1, NDR), jnp.int32),
      pltpu.VMEM_SHARED((NDR, D), jnp.float32),
  ]

  def body(dstr_hbm, deg_hbm, dst_v, deg_v, ridx_v, deg_sh):
    cid = lax.axis_index("c")
    sid = lax.axis_index("s")
    wid = cid * 16 + sid

    zero16 = jnp.zeros((16,), jnp.float32)

    @pl.loop(0, NDR)
    def _(r):
      @pl.loop(0, D, step=16)
      def _(c):
        deg_v[r, pl.ds(c, 16)] = zero16

    @pl.when(sid == 0)
    def _():
      pltpu.sync_copy(deg_v, deg_sh)
    iota16 = lax.iota(jnp.int32, 16)
    for j in range(NDR // 16):
      ridx_v[0, pl.ds(j * 16, 16)] = iota16 + j * 16

    plsc.subcore_barrier()

    pltpu.sync_copy(dstr_hbm.at[wid], dst_v)
    ones16 = jnp.ones((16,), jnp.float32)

    @pl.loop(0, n_chunks)
    def _(i):
      for j in range(CH // 16):
        idx = dst_v[i, pl.ds(j * 16, 16)]
        plsc.addupdate_scatter(
            deg_v,
            [lax.shift_right_logical(idx, 7), lax.bitwise_and(idx, 127)],
            ones16)

    pltpu.sync_copy(deg_v, deg_sh.at[ridx_v.at[0]], add=True)
    plsc.subcore_barrier()

    @pl.when(sid == 0)
    def _():
      pltpu.sync_copy(deg_sh, deg_hbm.at[pl.ds(cid * NDR, NDR)])

  return pl.kernel(
      body,
      out_type=jax.ShapeDtypeStruct((2 * NDR, D), jnp.float32),
      mesh=_mesh, scratch_types=scratch, compiler_params=_sc_params)


def _make_sc_agg(n_chunks):
  ng = 2 * n_chunks // KCH
  assert ng % NBUF == 0
  gw = KCH * CH
  scratch = [
      pltpu.VMEM((ng, gw), jnp.int32),
      pltpu.VMEM((ng, gw), jnp.int32),
  ]
  scratch += [pltpu.VMEM((gw, DH), jnp.float32) for _ in range(NBUF)]
  scratch += [
      pltpu.VMEM_SHARED((N_PAD, DH), jnp.float32),
  ]
  scratch += [pltpu.SemaphoreType.DMA for _ in range(2 * NBUF)]

  def body(h_hbm, srcr_hbm, dstr_hbm, acc_hbm, src_v, dst_v, *rest):
    bufs = rest[:NBUF]
    acc_sh = rest[NBUF]
    gsems = rest[NBUF + 1:NBUF + 1 + NBUF]
    ssems = rest[NBUF + 1 + NBUF:]
    cid = lax.axis_index("c")
    sid = lax.axis_index("s")

    zero16 = jnp.zeros((16,), jnp.float32)
    base = sid * ROWS_PER_TILE

    @pl.loop(0, gw)
    def _(r):
      @pl.loop(0, DH, step=16)
      def _(c):
        bufs[0][r, pl.ds(c, 16)] = zero16

    @pl.loop(0, ROWS_PER_TILE // gw)
    def _(k):
      pltpu.sync_copy(bufs[0], acc_sh.at[pl.ds(base + k * gw, gw)])

    plsc.subcore_barrier()

    nhalf = ng // 2

    @pl.loop(0, 2)
    def _(p):
      pltpu.sync_copy(srcr_hbm.at[cid, 2 * sid + p],
                      src_v.at[pl.ds(p * nhalf, nhalf)])
      pltpu.sync_copy(dstr_hbm.at[2 * sid + p],
                      dst_v.at[pl.ds(p * nhalf, nhalf)])

    def g_start(i, k):
      pltpu.async_copy(h_hbm.at[src_v.at[i]], bufs[k], gsems[k])

    def g_wait(i, k):
      pltpu.make_async_copy(h_hbm.at[src_v.at[i]], bufs[k], gsems[k]).wait()

    def s_start(i, k):
      pltpu.async_copy(bufs[k], acc_sh.at[dst_v.at[i]], ssems[k], add=True)

    def s_wait(i, k):
      pltpu.make_async_copy(bufs[k], acc_sh.at[dst_v.at[i]], ssems[k]).wait()

    @pl.loop(0, ng, step=NBUF)
    def _(i):
      for k in range(NBUF):
        @pl.when(i >= NBUF)
        def _():
          s_wait(i - NBUF + k, k)
        g_start(i + k, k)
      for k in range(NBUF):
        g_wait(i + k, k)
        s_start(i + k, k)

    for k in range(NBUF):
      s_wait(ng - NBUF + k, k)

    plsc.subcore_barrier()

    pltpu.sync_copy(acc_sh.at[pl.ds(base, ROWS_PER_TILE)],
                    acc_hbm.at[pl.ds(cid * N_PAD + base, ROWS_PER_TILE)])

  return pl.kernel(
      body,
      out_type=jax.ShapeDtypeStruct((2 * N_PAD, DH), jnp.float32),
      mesh=_mesh, scratch_types=scratch, compiler_params=_sc_params)


_DOT = functools.partial(
    lax.dot_general,
    dimension_numbers=(((1,), (0,)), ((), ())),
    preferred_element_type=jnp.float32)


def _k_in_body(x_ref, w_ref, b_ref, o_ref):
  t = jnp.tanh(_DOT(x_ref[...], w_ref[...]) + b_ref[...])
  o_ref[0] = t[:, :DH]
  o_ref[1] = t[:, DH:]


def _layer_math(h_ref, acc_ref, d0_ref, d1_ref, ws_ref, b_ref, wn_ref):
  deg = jnp.maximum(d0_ref[...] + d1_ref[...], 1.0)
  ws = ws_ref[...]
  wn = wn_ref[...]
  t = _DOT(h_ref[0], ws[:DH]) + _DOT(h_ref[1], ws[DH:])
  t += _DOT(acc_ref[0] / deg, wn[:DH]) + _DOT(acc_ref[1] / deg, wn[DH:])
  return jnp.maximum(t + b_ref[...], 0.0)


def _k_layer_body(h_ref, acc_ref, d0_ref, d1_ref, ws_ref, b_ref, wn_ref,
                  o_ref):
  r = _layer_math(h_ref, acc_ref, d0_ref, d1_ref, ws_ref, b_ref, wn_ref)
  o_ref[0] = r[:, :DH]
  o_ref[1] = r[:, DH:]


def _k_last_body(h_ref, acc_ref, deg3_ref, ws_ref, b_ref, wn_ref, o_ref):
  deg = jnp.maximum(deg3_ref[0] + deg3_ref[1], 1.0)
  ws = ws_ref[...]
  wn = wn_ref[...]
  t = _DOT(h_ref[0], ws[:DH]) + _DOT(h_ref[1], ws[DH:])
  t += _DOT(acc_ref[0] / deg, wn[:DH]) + _DOT(acc_ref[1] / deg, wn[DH:])
  o_ref[...] = jnp.maximum(t + b_ref[...], 0.0)


_BLK = 1024
_GRID = N_PAD // _BLK
_row_spec = pl.BlockSpec((_BLK, D), lambda i: (i, 0))
_split_spec = pl.BlockSpec((2, _BLK, DH), lambda i: (0, i, 0))
_w_spec = pl.BlockSpec((D, D), lambda i: (0, 0))
_b_spec = pl.BlockSpec((1, D), lambda i: (0, 0))
_split_sds = jax.ShapeDtypeStruct((2, N_PAD, DH), jnp.float32)

_k_in = pl.pallas_call(
    _k_in_body,
    grid=(_GRID,),
    in_specs=[_row_spec, _w_spec, _b_spec],
    out_specs=_split_spec,
    out_shape=_split_sds)

_layer_in_specs = [
    _split_spec,
    _split_spec,
    pl.BlockSpec((_BLK, 1), lambda i: (i, 0)),
    pl.BlockSpec((_BLK, 1), lambda i: (i + _GRID, 0)),
    _w_spec, _b_spec, _w_spec,
]

_k_layer = pl.pallas_call(
    _k_layer_body,
    grid=(_GRID,),
    in_specs=_layer_in_specs,
    out_specs=_split_spec,
    out_shape=_split_sds)

_LBLK = 1000
_k_last = pl.pallas_call(
    _k_last_body,
    grid=(N // _LBLK,),
    in_specs=[
        pl.BlockSpec((2, _LBLK, DH), lambda i: (0, i, 0)),
        pl.BlockSpec((2, _LBLK, DH), lambda i: (0, i, 0)),
        pl.BlockSpec((2, _LBLK, 1), lambda i: (0, i, 0)),
        _w_spec, _b_spec, _w_spec,
    ],
    out_specs=pl.BlockSpec((_LBLK, D), lambda i: (i, 0)),
    out_shape=jax.ShapeDtypeStruct((N, D), jnp.float32))


def kernel(x, edge_index, W_in, b_in, W_self1, b_self1, W_neigh1,
           W_self2, b_self2, W_neigh2):
  E = edge_index.shape[1]
  n_chunks = -(-E // (NW * CH))
  if n_chunks % 2:
    n_chunks += 1
  e_pad = NW * CH * n_chunks - E

  xp = jnp.zeros((N_PAD, D), jnp.float32).at[:N].set(x)
  src = edge_index[0]
  dst = edge_index[1]
  if e_pad:
    ar = jnp.arange(e_pad, dtype=jnp.int32)
    src = jnp.concatenate([src, ar % N])
    dst = jnp.concatenate([dst, N + ar % (N_PAD - N)])
  srcr = src.reshape(NW, n_chunks // KCH, KCH * CH)
  dstr = dst.reshape(NW, n_chunks // KCH, KCH * CH)
  srcr2 = jnp.stack([srcr, srcr + N_PAD])
  srcr2, dstr = lax.optimization_barrier((srcr2, dstr))

  sc_deg = _make_sc_deg(n_chunks)
  sc_agg = _make_sc_agg(n_chunks)

  b_in2 = b_in.reshape(1, D)
  b1 = b_self1.reshape(1, D)
  b2 = b_self2.reshape(1, D)

  deg = sc_deg(dstr.reshape(NW, n_chunks, CH))
  degf = deg.reshape(2 * N_PAD, 1)
  h0 = _k_in(xp, W_in, b_in2)
  acc1 = sc_agg(h0.reshape(2 * N_PAD, DH), srcr2, dstr)
  h1 = _k_layer(h0, acc1.reshape(2, N_PAD, DH), degf, degf, W_self1, b1,
                W_neigh1)
  acc2 = sc_agg(h1.reshape(2 * N_PAD, DH), srcr2, dstr)
  return _k_last(h1, acc2.reshape(2, N_PAD, DH), deg.reshape(2, N_PAD, 1),
                 W_self2, b2, W_neigh2)
